# baseline (speedup 1.0000x reference)
import jax
import jax.numpy as jnp
from jax.experimental import pallas as pl

N_ROWS = 16384
D = 64
K = 1024
BETA = 0.25
BLOCK = 4096
GRID = N_ROWS // BLOCK
AUG = 128


def _vq_block(x_ref, cb_ref, csqr_ref, cbaug_ref, zq_ref, idx_ref, loss_ref):
    i = pl.program_id(0)
    x = x_ref[...]
    cb = cb_ref[...]
    csqr = csqr_ref[...]

    scores = jax.lax.dot_general(
        x, cb, dimension_numbers=(((1,), (1,)), ((), ())),
        preferred_element_type=jnp.float32)
    fsqr = jnp.sum(x * x, axis=1, keepdims=True)
    dist = fsqr - 2.0 * scores + csqr

    min_d = jnp.min(dist, axis=1)

    eqf = (dist == min_d[:, None]).astype(jnp.bfloat16)
    qi = jax.lax.dot_general(
        eqf, cbaug_ref[...], dimension_numbers=(((1,), (0,)), ((), ())),
        preferred_element_type=jnp.float32)

    q = qi[:, :D]
    idx = (4.0 * qi[:, D] + qi[:, D + 1]).astype(jnp.int32)

    zq_ref[...] = x + (q - x)
    idx_ref[...] = idx.reshape(1, 1, BLOCK)

    part = jnp.sum(min_d).reshape(1, 1)

    @pl.when(i == 0)
    def _init():
        loss_ref[...] = jnp.zeros_like(loss_ref)

    loss_ref[...] += part


def kernel(inputs, codebook):
    x = inputs.reshape(N_ROWS, D)
    csqr = jnp.sum(codebook ** 2, axis=-1, keepdims=True).T
    j = jnp.arange(K, dtype=jnp.float32)
    iota_hi = (j // 4.0).reshape(K, 1)
    iota_lo = (j % 4.0).reshape(K, 1)
    cb_aug = jnp.concatenate(
        [codebook, iota_hi, iota_lo,
         jnp.zeros((K, AUG - D - 2), jnp.float32)], axis=1).astype(jnp.bfloat16)

    zq, idx3, loss_sum = pl.pallas_call(
        _vq_block,
        grid=(GRID,),
        in_specs=[
            pl.BlockSpec((BLOCK, D), lambda i: (i, 0)),
            pl.BlockSpec((K, D), lambda i: (0, 0)),
            pl.BlockSpec((1, K), lambda i: (0, 0)),
            pl.BlockSpec((K, AUG), lambda i: (0, 0)),
        ],
        out_specs=[
            pl.BlockSpec((BLOCK, D), lambda i: (i, 0)),
            pl.BlockSpec((1, 1, BLOCK), lambda i: (i, 0, 0)),
            pl.BlockSpec((1, 1), lambda i: (0, 0)),
        ],
        out_shape=[
            jax.ShapeDtypeStruct((N_ROWS, D), jnp.float32),
            jax.ShapeDtypeStruct((GRID, 1, BLOCK), jnp.int32),
            jax.ShapeDtypeStruct((1, 1), jnp.float32),
        ],
    )(x, codebook, csqr, cb_aug)

    loss = loss_sum[0, 0] * ((1.0 + BETA) / (N_ROWS * D))
    z_q = zq.reshape(inputs.shape)
    encoding_indices = idx3.reshape(inputs.shape[:-1])
    return (loss, z_q, encoding_indices)


# speedup vs baseline: 1.0863x; 1.0863x over previous
import jax
import jax.numpy as jnp
from jax.experimental import pallas as pl

N_ROWS = 16384
D = 64
K = 1024
BETA = 0.25
BLOCK = 4096
GRID = N_ROWS // BLOCK
AUG = 128


def _vq_block(x_ref, cb_ref, csqr_ref, cbaug_ref, zq_ref, idx_ref, loss_ref):
    i = pl.program_id(0)
    x = x_ref[...]
    cb = cb_ref[...]
    csqr = csqr_ref[...]

    scores = jax.lax.dot_general(
        x, cb, dimension_numbers=(((1,), (1,)), ((), ())),
        preferred_element_type=jnp.float32)
    fsqr = jnp.sum(x * x, axis=1, keepdims=True)
    dist = fsqr - 2.0 * scores + csqr

    min_d = jnp.min(dist, axis=1)

    eqf = (dist == min_d[:, None]).astype(jnp.float32)
    qi = jax.lax.dot_general(
        eqf, cbaug_ref[...], dimension_numbers=(((1,), (0,)), ((), ())),
        preferred_element_type=jnp.float32)

    q = qi[:, :D]
    idx = (4.0 * qi[:, D] + qi[:, D + 1]).astype(jnp.int32)

    zq_ref[...] = x + (q - x)
    idx_ref[...] = idx.reshape(1, 1, BLOCK)

    part = jnp.sum(min_d).reshape(1, 1)

    @pl.when(i == 0)
    def _init():
        loss_ref[...] = jnp.zeros_like(loss_ref)

    loss_ref[...] += part


def kernel(inputs, codebook):
    x = inputs.reshape(N_ROWS, D)
    csqr = jnp.sum(codebook ** 2, axis=-1, keepdims=True).T
    j = jnp.arange(K, dtype=jnp.float32)
    iota_hi = (j // 4.0).reshape(K, 1)
    iota_lo = (j % 4.0).reshape(K, 1)
    cb_aug = jnp.concatenate(
        [codebook, iota_hi, iota_lo,
         jnp.zeros((K, AUG - D - 2), jnp.float32)], axis=1)

    zq, idx3, loss_sum = pl.pallas_call(
        _vq_block,
        grid=(GRID,),
        in_specs=[
            pl.BlockSpec((BLOCK, D), lambda i: (i, 0)),
            pl.BlockSpec((K, D), lambda i: (0, 0)),
            pl.BlockSpec((1, K), lambda i: (0, 0)),
            pl.BlockSpec((K, AUG), lambda i: (0, 0)),
        ],
        out_specs=[
            pl.BlockSpec((BLOCK, D), lambda i: (i, 0)),
            pl.BlockSpec((1, 1, BLOCK), lambda i: (i, 0, 0)),
            pl.BlockSpec((1, 1), lambda i: (0, 0)),
        ],
        out_shape=[
            jax.ShapeDtypeStruct((N_ROWS, D), jnp.float32),
            jax.ShapeDtypeStruct((GRID, 1, BLOCK), jnp.int32),
            jax.ShapeDtypeStruct((1, 1), jnp.float32),
        ],
    )(x, codebook, csqr, cb_aug)

    loss = loss_sum[0, 0] * ((1.0 + BETA) / (N_ROWS * D))
    z_q = zq.reshape(inputs.shape)
    encoding_indices = idx3.reshape(inputs.shape[:-1])
    return (loss, z_q, encoding_indices)
